# baseline (speedup 1.0000x reference)
import jax
import jax.numpy as jnp
from jax.experimental import pallas as pl
from jax.experimental.pallas import tpu as pltpu

B, N, E = 16, 256, 512
NODE_IN, EDGE_IN, NODE_OUT, EDGE_OUT = 128, 16, 128, 16


def _dot(a, b):
    return jnp.dot(a, b, preferred_element_type=jnp.float32)


def _kernel(node_ref, edge_ref, node_adj_ref, edge_adj_ref, D_v_ref, D_e_ref,
            T_ref, Wn1_ref, We1_ref, pe1_ref, pv1_ref, Wn2_ref, We2_ref,
            pe2_ref, pv2_ref, Wv1_ref, bv1_ref, Wv2_ref, bv2_ref, out_ref):
    n = node_ref[0]
    e = edge_ref[0]
    Av = node_adj_ref[0]
    Ae = edge_adj_ref[0]
    Dv = D_v_ref[0]
    De = D_e_ref[0]
    Tm = T_ref[0]

    def censnet(n, e, Wn, We, pe, pv):
        de = _dot(e, pe)
        Tde = Tm * de[:, 0][None, :]
        A_node = jax.lax.dot_general(
            Tde, Tm, (((1,), (1,)), ((), ())),
            preferred_element_type=jnp.float32) * Av
        x = _dot(n, Wn)
        x = _dot(Dv, x)
        x = _dot(A_node, x)
        n_new = jax.nn.relu(_dot(Dv, x))
        dv = _dot(n, pv)
        Tdv = Tm * dv
        A_edge = jax.lax.dot_general(
            Tdv, Tm, (((0,), (0,)), ((), ())),
            preferred_element_type=jnp.float32) * Ae
        y = _dot(e, We)
        y = _dot(De, y)
        y = _dot(A_edge, y)
        e_new = jax.nn.relu(_dot(De, y))
        return n_new, e_new

    n, e = censnet(n, e, Wn1_ref[...], We1_ref[...], pe1_ref[...], pv1_ref[...])
    n, e = censnet(n, e, Wn2_ref[...], We2_ref[...], pe2_ref[...], pv2_ref[...])

    v = jax.nn.relu(_dot(n, Wv1_ref[...]) + bv1_ref[...][None, :])
    vm = jnp.mean(v, axis=0, keepdims=True)
    out_ref[0] = _dot(vm, Wv2_ref[...]) + bv2_ref[...][None, :]


def kernel(node, edge, node_adj, edge_adj, D_v, D_e, T,
           Wn1, We1, pe1, pv1, Wn2, We2, pe2, pv2,
           Wv1, bv1, Wv2, bv2):
    batch = lambda *dims: pl.BlockSpec((1,) + dims, lambda b: (b, 0, 0))
    full = lambda arr: pl.BlockSpec(arr.shape, lambda b: (0,) * arr.ndim)
    grid_spec = pl.GridSpec(
        grid=(B,),
        in_specs=[
            batch(N, NODE_IN),
            batch(E, EDGE_IN),
            batch(N, N),
            batch(E, E),
            batch(N, N),
            batch(E, E),
            batch(N, E),
            full(Wn1), full(We1), full(pe1), full(pv1),
            full(Wn2), full(We2), full(pe2), full(pv2),
            full(Wv1), full(bv1), full(Wv2), full(bv2),
        ],
        out_specs=pl.BlockSpec((1, 1, 1), lambda b: (b, 0, 0)),
    )
    out = pl.pallas_call(
        _kernel,
        grid_spec=grid_spec,
        out_shape=jax.ShapeDtypeStruct((B, 1, 1), jnp.float32),
        compiler_params=pltpu.CompilerParams(
            dimension_semantics=("parallel",),
        ),
    )(node, edge, node_adj, edge_adj, D_v, D_e, T,
      Wn1, We1, pe1, pv1, Wn2, We2, pe2, pv2,
      Wv1, bv1, Wv2, bv2)
    return out.reshape(B, 1)


# speedup vs baseline: 1.3134x; 1.3134x over previous
import jax
import jax.numpy as jnp
from jax.experimental import pallas as pl
from jax.experimental.pallas import tpu as pltpu

B, N, E = 16, 256, 512
NODE_IN, EDGE_IN, NODE_OUT, EDGE_OUT = 128, 16, 128, 16

_F32 = jnp.float32


def _dot(a, b):
    return jnp.dot(a, b, preferred_element_type=_F32)


def _dg(a, b, dims):
    return jax.lax.dot_general(a, b, (dims, ((), ())),
                               preferred_element_type=_F32)


def _kernel(node_ref, edge_ref, node_adj_ref, edge_adj_ref, D_v_ref, D_e_ref,
            T_ref, Wn1_ref, We1_ref, pe1_ref, pv1_ref, Wn2_ref, We2_ref,
            pe2_ref, pv2_ref, Wv1_ref, bv1_ref, Wv2_ref, bv2_ref, out_ref):
    n = node_ref[0]
    e = edge_ref[0]
    Av = node_adj_ref[0]
    Ae = edge_adj_ref[0]
    Dv = D_v_ref[0]
    De = D_e_ref[0]
    Tm = T_ref[0]

    def node_prop(n, deT, Wn):
        Tde = Tm * deT
        A_node = _dg(Tde, Tm, ((1,), (1,))) * Av
        x = _dot(n, Wn)
        x = _dot(Dv, x)
        x = _dot(A_node, x)
        return jax.nn.relu(_dot(Dv, x))

    de1T = _dg(pe1_ref[...], e, ((0,), (1,)))
    n1 = node_prop(n, de1T, Wn1_ref[...])

    dv1 = _dot(n, pv1_ref[...])
    Tdv = Tm * dv1
    A_edge = _dg(Tdv, Tm, ((0,), (0,))) * Ae
    yT = _dg(We1_ref[...], e, ((0,), (1,)))
    yT = _dg(yT, De, ((1,), (1,)))
    yT = _dg(yT, A_edge, ((1,), (1,)))
    e1T = jax.nn.relu(_dg(yT, De, ((1,), (1,))))

    de2T = _dg(pe2_ref[...], e1T, ((0,), (0,)))
    n2 = node_prop(n1, de2T, Wn2_ref[...])

    v = jax.nn.relu(_dot(n2, Wv1_ref[...]) + bv1_ref[...][None, :])
    vm = jnp.mean(v, axis=0, keepdims=True)
    out_ref[0] = _dot(vm, Wv2_ref[...]) + bv2_ref[...][None, :]


def kernel(node, edge, node_adj, edge_adj, D_v, D_e, T,
           Wn1, We1, pe1, pv1, Wn2, We2, pe2, pv2,
           Wv1, bv1, Wv2, bv2):
    batch = lambda *dims: pl.BlockSpec((1,) + dims, lambda b: (b, 0, 0))
    full = lambda arr: pl.BlockSpec(arr.shape, lambda b: (0,) * arr.ndim)
    grid_spec = pl.GridSpec(
        grid=(B,),
        in_specs=[
            batch(N, NODE_IN),
            batch(E, EDGE_IN),
            batch(N, N),
            batch(E, E),
            batch(N, N),
            batch(E, E),
            batch(N, E),
            full(Wn1), full(We1), full(pe1), full(pv1),
            full(Wn2), full(We2), full(pe2), full(pv2),
            full(Wv1), full(bv1), full(Wv2), full(bv2),
        ],
        out_specs=pl.BlockSpec((1, 1, 1), lambda b: (b, 0, 0)),
    )
    out = pl.pallas_call(
        _kernel,
        grid_spec=grid_spec,
        out_shape=jax.ShapeDtypeStruct((B, 1, 1), jnp.float32),
        compiler_params=pltpu.CompilerParams(
            dimension_semantics=("parallel",),
        ),
    )(node, edge, node_adj, edge_adj, D_v, D_e, T,
      Wn1, We1, pe1, pv1, Wn2, We2, pe2, pv2,
      Wv1, bv1, Wv2, bv2)
    return out.reshape(B, 1)
